# baseline (speedup 1.0000x reference)
import functools

import jax
import jax.numpy as jnp
from jax import lax
from jax.experimental import pallas as pl
from jax.experimental.pallas import tpu as pltpu
from jax.experimental.pallas import tpu_sc as plsc

MARGIN = 1e-05

N_SAMPLE = 48
P = N_SAMPLE * N_SAMPLE
N_CLASSES = 1000

NUM_WORKERS = 16
PER_W = P // NUM_WORKERS
CHUNK = PER_W // 2
ROW_BLK = 128


def _sc_gather_body(labels_hbm, csm_hbm, out_hbm,
                    labels_v, rix_v, li_v, idx_v, vals_v, sem):
    wid = lax.axis_index("s")
    base = wid * PER_W
    ROWS_W = PER_W // N_SAMPLE
    NCH = N_SAMPLE // 16

    lab_cp = pltpu.async_copy(labels_hbm, labels_v, sem)

    for r in range(ROWS_W):
        i = ROWS_W * wid + r
        ivec = jnp.full((16,), i, jnp.int32)
        for c in range(NCH):
            rix_v[pl.ds((r * NCH + c) * 16, 16)] = ivec

    cps = [pltpu.async_copy(labels_hbm.at[rix_v.at[pl.ds(c * CHUNK, CHUNK)]],
                            li_v.at[pl.ds(c * CHUNK, CHUNK)], sem)
           for c in range(2)]
    lab_cp.wait()
    for cp in cps:
        cp.wait()

    for r in range(ROWS_W):
        for c in range(NCH):
            sl = pl.ds((r * NCH + c) * 16, 16)
            idx_v[sl] = li_v[sl] * N_CLASSES + labels_v[pl.ds(c * 16, 16)]

    cps = [pltpu.async_copy(csm_hbm.at[idx_v.at[pl.ds(c * CHUNK, CHUNK)]],
                            vals_v.at[pl.ds(c * CHUNK, CHUNK)], sem)
           for c in range(2)]
    for cp in cps:
        cp.wait()
    pltpu.sync_copy(vals_v, out_hbm.at[pl.ds(base, PER_W)])


@functools.partial(jax.jit, static_argnames=())
def _sc_gather(labels, csm_flat):
    mesh = plsc.VectorSubcoreMesh(core_axis_name="c", subcore_axis_name="s",
                                  num_cores=1)
    kern = functools.partial(
        pl.kernel,
        out_type=jax.ShapeDtypeStruct((P,), jnp.float32),
        mesh=mesh,
        scratch_types=[
            pltpu.VMEM((N_SAMPLE,), jnp.int32),
            pltpu.VMEM((PER_W,), jnp.int32),
            pltpu.VMEM((PER_W,), jnp.int32),
            pltpu.VMEM((PER_W,), jnp.int32),
            pltpu.VMEM((PER_W,), jnp.float32),
            pltpu.SemaphoreType.DMA,
        ],
    )(_sc_gather_body)
    return kern(labels, csm_flat)


def _pair_loss_body(acol, bcol, arow, brow, out):
    @pl.when(pl.program_id(0) == 0)
    def _init():
        out[...] = jnp.zeros((1, 1), jnp.float32)

    d = arow[...] - acol[...]
    bd = bcol[...] - brow[...]
    s = jnp.sign(bd)
    t = jnp.where(bd != 0.0,
                  jnp.maximum(s * d + MARGIN, 0.0),
                  jnp.abs(d))
    out[...] += jnp.sum(t).reshape(1, 1)


def _pair_loss(a, b):
    acol = a.reshape(P, 1)
    bcol = b.reshape(P, 1)
    arow = a.reshape(1, P)
    brow = b.reshape(1, P)
    tot = pl.pallas_call(
        _pair_loss_body,
        grid=(P // ROW_BLK,),
        in_specs=[
            pl.BlockSpec((ROW_BLK, 1), lambda i: (i, 0)),
            pl.BlockSpec((ROW_BLK, 1), lambda i: (i, 0)),
            pl.BlockSpec((1, P), lambda i: (0, 0)),
            pl.BlockSpec((1, P), lambda i: (0, 0)),
        ],
        out_specs=pl.BlockSpec((1, 1), lambda i: (0, 0)),
        out_shape=jax.ShapeDtypeStruct((1, 1), jnp.float32),
    )(acol, bcol, arow, brow)
    return tot[0, 0]


def kernel(FSR_Mat, labels, class_sim_mat):
    labels = labels.astype(jnp.int32)
    csm_flat = class_sim_mat.reshape(-1)
    a = FSR_Mat.reshape(-1)
    b = _sc_gather(labels, csm_flat)
    total = _pair_loss(a, b)
    n4 = float(P) * float(P)
    return total / n4


# speedup vs baseline: 1.0001x; 1.0001x over previous
import functools

import jax
import jax.numpy as jnp
from jax import lax
from jax.experimental import pallas as pl
from jax.experimental.pallas import tpu as pltpu
from jax.experimental.pallas import tpu_sc as plsc

MARGIN = 1e-05

N_SAMPLE = 48
P = N_SAMPLE * N_SAMPLE
N_CLASSES = 1000

NUM_WORKERS = 16
PER_W = P // NUM_WORKERS
CHUNK = PER_W // 2
ROW_BLK = 128


def _sc_gather_body(labels_hbm, csm_hbm, out_hbm,
                    labels_v, rix_v, li_v, idx_v, vals_v, sem):
    wid = lax.axis_index("s")
    base = wid * PER_W
    ROWS_W = PER_W // N_SAMPLE
    NCH = N_SAMPLE // 16

    lab_cp = pltpu.async_copy(labels_hbm, labels_v, sem)

    for r in range(ROWS_W):
        i = ROWS_W * wid + r
        ivec = jnp.full((16,), i, jnp.int32)
        for c in range(NCH):
            rix_v[pl.ds((r * NCH + c) * 16, 16)] = ivec

    cp = pltpu.async_copy(labels_hbm.at[rix_v], li_v, sem)
    lab_cp.wait()
    cp.wait()

    for r in range(ROWS_W):
        for c in range(NCH):
            sl = pl.ds((r * NCH + c) * 16, 16)
            idx_v[sl] = li_v[sl] * N_CLASSES + labels_v[pl.ds(c * 16, 16)]

    pltpu.async_copy(csm_hbm.at[idx_v], vals_v, sem).wait()
    pltpu.sync_copy(vals_v, out_hbm.at[pl.ds(base, PER_W)])


@functools.partial(jax.jit, static_argnames=())
def _sc_gather(labels, csm_flat):
    mesh = plsc.VectorSubcoreMesh(core_axis_name="c", subcore_axis_name="s",
                                  num_cores=1)
    kern = functools.partial(
        pl.kernel,
        out_type=jax.ShapeDtypeStruct((P,), jnp.float32),
        mesh=mesh,
        scratch_types=[
            pltpu.VMEM((N_SAMPLE,), jnp.int32),
            pltpu.VMEM((PER_W,), jnp.int32),
            pltpu.VMEM((PER_W,), jnp.int32),
            pltpu.VMEM((PER_W,), jnp.int32),
            pltpu.VMEM((PER_W,), jnp.float32),
            pltpu.SemaphoreType.DMA,
        ],
    )(_sc_gather_body)
    return kern(labels, csm_flat)


def _pair_loss_body(acol, bcol, arow, brow, out):
    @pl.when(pl.program_id(0) == 0)
    def _init():
        out[...] = jnp.zeros((1, 1), jnp.float32)

    d = arow[...] - acol[...]
    bd = bcol[...] - brow[...]
    s = jnp.sign(bd)
    t = jnp.where(bd != 0.0,
                  jnp.maximum(s * d + MARGIN, 0.0),
                  jnp.abs(d))
    out[...] += jnp.sum(t).reshape(1, 1)


def _pair_loss(a, b):
    acol = a.reshape(P, 1)
    bcol = b.reshape(P, 1)
    arow = a.reshape(1, P)
    brow = b.reshape(1, P)
    tot = pl.pallas_call(
        _pair_loss_body,
        grid=(P // ROW_BLK,),
        in_specs=[
            pl.BlockSpec((ROW_BLK, 1), lambda i: (i, 0)),
            pl.BlockSpec((ROW_BLK, 1), lambda i: (i, 0)),
            pl.BlockSpec((1, P), lambda i: (0, 0)),
            pl.BlockSpec((1, P), lambda i: (0, 0)),
        ],
        out_specs=pl.BlockSpec((1, 1), lambda i: (0, 0)),
        out_shape=jax.ShapeDtypeStruct((1, 1), jnp.float32),
    )(acol, bcol, arow, brow)
    return tot[0, 0]


def kernel(FSR_Mat, labels, class_sim_mat):
    labels = labels.astype(jnp.int32)
    csm_flat = class_sim_mat.reshape(-1)
    a = FSR_Mat.reshape(-1)
    b = _sc_gather(labels, csm_flat)
    total = _pair_loss(a, b)
    n4 = float(P) * float(P)
    return total / n4


# speedup vs baseline: 1.2357x; 1.2356x over previous
import functools

import jax
import jax.numpy as jnp
from jax import lax
from jax.experimental import pallas as pl
from jax.experimental.pallas import tpu as pltpu
from jax.experimental.pallas import tpu_sc as plsc

MARGIN = 1e-05

N_SAMPLE = 48
P = N_SAMPLE * N_SAMPLE
N_CLASSES = 1000

NUM_WORKERS = 16
PER_W = P // NUM_WORKERS
CHUNK = PER_W // 2
ROW_BLK = 128


def _sc_gather_body(idx_hbm, csm_hbm, out_hbm, idx_v, vals_v, sem):
    wid = lax.axis_index("s")
    base = wid * PER_W

    pltpu.sync_copy(idx_hbm.at[pl.ds(base, PER_W)], idx_v)
    pltpu.async_copy(csm_hbm.at[idx_v], vals_v, sem).wait()
    pltpu.sync_copy(vals_v, out_hbm.at[pl.ds(base, PER_W)])


@functools.partial(jax.jit, static_argnames=())
def _sc_gather(idx, csm_flat):
    mesh = plsc.VectorSubcoreMesh(core_axis_name="c", subcore_axis_name="s",
                                  num_cores=1)
    kern = functools.partial(
        pl.kernel,
        out_type=jax.ShapeDtypeStruct((P,), jnp.float32),
        mesh=mesh,
        scratch_types=[
            pltpu.VMEM((PER_W,), jnp.int32),
            pltpu.VMEM((PER_W,), jnp.float32),
            pltpu.SemaphoreType.DMA,
        ],
    )(_sc_gather_body)
    return kern(idx, csm_flat)


def _pair_loss_body(acol, bcol, arow, brow, out):
    @pl.when(pl.program_id(0) == 0)
    def _init():
        out[...] = jnp.zeros((1, 1), jnp.float32)

    d = arow[...] - acol[...]
    bd = bcol[...] - brow[...]
    s = jnp.sign(bd)
    t = jnp.where(bd != 0.0,
                  jnp.maximum(s * d + MARGIN, 0.0),
                  jnp.abs(d))
    out[...] += jnp.sum(t).reshape(1, 1)


def _pair_loss(a, b):
    acol = a.reshape(P, 1)
    bcol = b.reshape(P, 1)
    arow = a.reshape(1, P)
    brow = b.reshape(1, P)
    tot = pl.pallas_call(
        _pair_loss_body,
        grid=(P // ROW_BLK,),
        in_specs=[
            pl.BlockSpec((ROW_BLK, 1), lambda i: (i, 0)),
            pl.BlockSpec((ROW_BLK, 1), lambda i: (i, 0)),
            pl.BlockSpec((1, P), lambda i: (0, 0)),
            pl.BlockSpec((1, P), lambda i: (0, 0)),
        ],
        out_specs=pl.BlockSpec((1, 1), lambda i: (0, 0)),
        out_shape=jax.ShapeDtypeStruct((1, 1), jnp.float32),
    )(acol, bcol, arow, brow)
    return tot[0, 0]


def kernel(FSR_Mat, labels, class_sim_mat):
    labels = labels.astype(jnp.int32)
    csm_flat = class_sim_mat.reshape(-1)
    a = FSR_Mat.reshape(-1)
    idx = (labels * N_CLASSES)[:, None] + labels[None, :]
    b = _sc_gather(idx.reshape(-1), csm_flat)
    total = _pair_loss(a, b)
    n4 = float(P) * float(P)
    return total / n4


# speedup vs baseline: 1.4244x; 1.1527x over previous
import functools

import jax
import jax.numpy as jnp
from jax import lax
from jax.experimental import pallas as pl
from jax.experimental.pallas import tpu as pltpu
from jax.experimental.pallas import tpu_sc as plsc

MARGIN = 1e-05

N_SAMPLE = 48
P = N_SAMPLE * N_SAMPLE
N_CLASSES = 1000

NUM_WORKERS = 16
PER_W = P // NUM_WORKERS
CHUNK = PER_W // 2
ROW_BLK = 384


def _sc_gather_body(idx_hbm, csm_hbm, out_hbm, idx_v, vals_v, sem):
    wid = lax.axis_index("s")
    base = wid * PER_W

    pltpu.sync_copy(idx_hbm.at[pl.ds(base, PER_W)], idx_v)
    pltpu.async_copy(csm_hbm.at[idx_v], vals_v, sem).wait()
    pltpu.sync_copy(vals_v, out_hbm.at[pl.ds(base, PER_W)])


@functools.partial(jax.jit, static_argnames=())
def _sc_gather(idx, csm_flat):
    mesh = plsc.VectorSubcoreMesh(core_axis_name="c", subcore_axis_name="s",
                                  num_cores=1)
    kern = functools.partial(
        pl.kernel,
        out_type=jax.ShapeDtypeStruct((P,), jnp.float32),
        mesh=mesh,
        scratch_types=[
            pltpu.VMEM((PER_W,), jnp.int32),
            pltpu.VMEM((PER_W,), jnp.float32),
            pltpu.SemaphoreType.DMA,
        ],
    )(_sc_gather_body)
    return kern(idx, csm_flat)


def _pair_loss_body(acol, bcol, arow, brow, out):
    @pl.when(pl.program_id(0) == 0)
    def _init():
        out[...] = jnp.zeros((1, 1), jnp.float32)

    d = arow[...] - acol[...]
    bd = bcol[...] - brow[...]
    u = jnp.where(bd < 0.0, -d, d)
    t = jnp.where(bd == 0.0,
                  jnp.abs(u),
                  jnp.maximum(u + MARGIN, 0.0))
    out[...] += jnp.sum(t).reshape(1, 1)


def _pair_loss(a, b):
    acol = a.reshape(P, 1)
    bcol = b.reshape(P, 1)
    arow = a.reshape(1, P)
    brow = b.reshape(1, P)
    tot = pl.pallas_call(
        _pair_loss_body,
        grid=(P // ROW_BLK,),
        in_specs=[
            pl.BlockSpec((ROW_BLK, 1), lambda i: (i, 0)),
            pl.BlockSpec((ROW_BLK, 1), lambda i: (i, 0)),
            pl.BlockSpec((1, P), lambda i: (0, 0)),
            pl.BlockSpec((1, P), lambda i: (0, 0)),
        ],
        out_specs=pl.BlockSpec((1, 1), lambda i: (0, 0)),
        out_shape=jax.ShapeDtypeStruct((1, 1), jnp.float32),
    )(acol, bcol, arow, brow)
    return tot[0, 0]


def kernel(FSR_Mat, labels, class_sim_mat):
    labels = labels.astype(jnp.int32)
    csm_flat = class_sim_mat.reshape(-1)
    a = FSR_Mat.reshape(-1)
    idx = (labels * N_CLASSES)[:, None] + labels[None, :]
    b = _sc_gather(idx.reshape(-1), csm_flat)
    total = _pair_loss(a, b)
    n4 = float(P) * float(P)
    return total / n4


# speedup vs baseline: 1.4435x; 1.0134x over previous
import functools

import jax
import jax.numpy as jnp
from jax import lax
from jax.experimental import pallas as pl
from jax.experimental.pallas import tpu as pltpu
from jax.experimental.pallas import tpu_sc as plsc

MARGIN = 1e-05

N_SAMPLE = 48
P = N_SAMPLE * N_SAMPLE
N_CLASSES = 1000

NUM_WORKERS = 16
PER_W = P // NUM_WORKERS
CHUNK = PER_W // 2
ROW_BLK = 1152


def _sc_gather_body(idx_hbm, csm_hbm, out_hbm, idx_v, vals_v, sem):
    wid = lax.axis_index("s")
    base = wid * PER_W

    pltpu.sync_copy(idx_hbm.at[pl.ds(base, PER_W)], idx_v)
    pltpu.async_copy(csm_hbm.at[idx_v], vals_v, sem).wait()
    pltpu.sync_copy(vals_v, out_hbm.at[pl.ds(base, PER_W)])


@functools.partial(jax.jit, static_argnames=())
def _sc_gather(idx, csm_flat):
    mesh = plsc.VectorSubcoreMesh(core_axis_name="c", subcore_axis_name="s",
                                  num_cores=1)
    kern = functools.partial(
        pl.kernel,
        out_type=jax.ShapeDtypeStruct((P,), jnp.float32),
        mesh=mesh,
        scratch_types=[
            pltpu.VMEM((PER_W,), jnp.int32),
            pltpu.VMEM((PER_W,), jnp.float32),
            pltpu.SemaphoreType.DMA,
        ],
    )(_sc_gather_body)
    return kern(idx, csm_flat)


def _pair_loss_body(acol, bcol, arow, brow, out):
    @pl.when(pl.program_id(0) == 0)
    def _init():
        out[...] = jnp.zeros((1, 1), jnp.float32)

    d = arow[...] - acol[...]
    bd = bcol[...] - brow[...]
    u = jnp.where(bd < 0.0, -d, d)
    t = jnp.where(bd == 0.0,
                  jnp.abs(u),
                  jnp.maximum(u + MARGIN, 0.0))
    out[...] += jnp.sum(t).reshape(1, 1)


def _pair_loss(a, b):
    acol = a.reshape(P, 1)
    bcol = b.reshape(P, 1)
    arow = a.reshape(1, P)
    brow = b.reshape(1, P)
    tot = pl.pallas_call(
        _pair_loss_body,
        grid=(P // ROW_BLK,),
        in_specs=[
            pl.BlockSpec((ROW_BLK, 1), lambda i: (i, 0)),
            pl.BlockSpec((ROW_BLK, 1), lambda i: (i, 0)),
            pl.BlockSpec((1, P), lambda i: (0, 0)),
            pl.BlockSpec((1, P), lambda i: (0, 0)),
        ],
        out_specs=pl.BlockSpec((1, 1), lambda i: (0, 0)),
        out_shape=jax.ShapeDtypeStruct((1, 1), jnp.float32),
    )(acol, bcol, arow, brow)
    return tot[0, 0]


def kernel(FSR_Mat, labels, class_sim_mat):
    labels = labels.astype(jnp.int32)
    csm_flat = class_sim_mat.reshape(-1)
    a = FSR_Mat.reshape(-1)
    idx = (labels * N_CLASSES)[:, None] + labels[None, :]
    b = _sc_gather(idx.reshape(-1), csm_flat)
    total = _pair_loss(a, b)
    n4 = float(P) * float(P)
    return total / n4


# speedup vs baseline: 1.4915x; 1.0333x over previous
import functools

import jax
import jax.numpy as jnp
from jax import lax
from jax.experimental import pallas as pl
from jax.experimental.pallas import tpu as pltpu
from jax.experimental.pallas import tpu_sc as plsc

MARGIN = 1e-05

N_SAMPLE = 48
P = N_SAMPLE * N_SAMPLE
N_CLASSES = 1000

NUM_WORKERS = 16
PER_W = P // NUM_WORKERS
CHUNK = PER_W // 2
ROW_BLK = 2304


def _sc_gather_body(idx_hbm, csm_hbm, out_hbm, idx_v, vals_v, sem):
    wid = lax.axis_index("s")
    base = wid * PER_W

    pltpu.sync_copy(idx_hbm.at[pl.ds(base, PER_W)], idx_v)
    pltpu.async_copy(csm_hbm.at[idx_v], vals_v, sem).wait()
    pltpu.sync_copy(vals_v, out_hbm.at[pl.ds(base, PER_W)])


@functools.partial(jax.jit, static_argnames=())
def _sc_gather(idx, csm_flat):
    mesh = plsc.VectorSubcoreMesh(core_axis_name="c", subcore_axis_name="s",
                                  num_cores=1)
    kern = functools.partial(
        pl.kernel,
        out_type=jax.ShapeDtypeStruct((P,), jnp.float32),
        mesh=mesh,
        scratch_types=[
            pltpu.VMEM((PER_W,), jnp.int32),
            pltpu.VMEM((PER_W,), jnp.float32),
            pltpu.SemaphoreType.DMA,
        ],
    )(_sc_gather_body)
    return kern(idx, csm_flat)


def _pair_loss_body(acol, bcol, arow, brow, out):
    @pl.when(pl.program_id(0) == 0)
    def _init():
        out[...] = jnp.zeros((1, 1), jnp.float32)

    d = arow[...] - acol[...]
    bd = bcol[...] - brow[...]
    u = jnp.where(bd < 0.0, -d, d)
    t = jnp.where(bd == 0.0,
                  jnp.abs(u),
                  jnp.maximum(u + MARGIN, 0.0))
    out[...] += jnp.sum(t).reshape(1, 1) * (1.0 / (float(P) * float(P)))


def _pair_loss(a, b):
    acol = a.reshape(P, 1)
    bcol = b.reshape(P, 1)
    arow = a.reshape(1, P)
    brow = b.reshape(1, P)
    tot = pl.pallas_call(
        _pair_loss_body,
        grid=(P // ROW_BLK,),
        in_specs=[
            pl.BlockSpec((ROW_BLK, 1), lambda i: (i, 0)),
            pl.BlockSpec((ROW_BLK, 1), lambda i: (i, 0)),
            pl.BlockSpec((1, P), lambda i: (0, 0)),
            pl.BlockSpec((1, P), lambda i: (0, 0)),
        ],
        out_specs=pl.BlockSpec((1, 1), lambda i: (0, 0)),
        out_shape=jax.ShapeDtypeStruct((1, 1), jnp.float32),
    )(acol, bcol, arow, brow)
    return tot[0, 0]


def kernel(FSR_Mat, labels, class_sim_mat):
    labels = labels.astype(jnp.int32)
    csm_flat = class_sim_mat.reshape(-1)
    a = FSR_Mat.reshape(-1)
    idx = (labels * N_CLASSES)[:, None] + labels[None, :]
    b = _sc_gather(idx.reshape(-1), csm_flat)
    return _pair_loss(a, b)
